# Initial kernel scaffold; baseline (speedup 1.0000x reference)
import jax
import jax.numpy as jnp
from jax.experimental import pallas as pl


def kernel(bbox, x_emb, y_emb, w_emb, h_emb):
    raise NotImplementedError("write your pallas kernel here")


# speedup vs baseline: 3.4848x; 3.4848x over previous
import functools

import jax
import jax.numpy as jnp
from jax import lax
from jax.experimental import pallas as pl
from jax.experimental.pallas import tpu as pltpu
from jax.experimental.pallas import tpu_sc as plsc

NUM_BINS = 1000
OUT_DIM = 64
B, S = 4096, 200
TOTAL = B * S * 4
NC, NS = 2, 16
NW = NC * NS
PER_W = TOTAL // NW
G = 128
STEPS = PER_W // G


def _body(idx_hbm, table_hbm, out_hbm, idx_v, rows_v, sem):
    wid = lax.axis_index("s") * NC + lax.axis_index("c")
    base = wid * PER_W
    offs = lax.rem(lax.iota(jnp.int32, 16), 4) * NUM_BINS

    def step(t, carry):
        row0 = base + t * G
        pltpu.sync_copy(idx_hbm.at[pl.ds(row0, G)], idx_v)
        for j in range(G // 16):
            sl = pl.ds(j * 16, 16)
            idx_v[sl] = idx_v[sl] + offs
        pltpu.async_copy(table_hbm.at[idx_v], rows_v, sem).wait()
        pltpu.sync_copy(rows_v, out_hbm.at[pl.ds(row0, G)])
        return carry

    lax.fori_loop(0, STEPS, step, 0)


@functools.partial(jax.jit, donate_argnums=())
def _gather(idx, table):
    mesh = plsc.VectorSubcoreMesh(core_axis_name="c", subcore_axis_name="s")
    return pl.kernel(
        _body,
        out_type=jax.ShapeDtypeStruct((TOTAL, OUT_DIM), jnp.float32),
        mesh=mesh,
        compiler_params=pltpu.CompilerParams(use_tc_tiling_on_sc=False),
        scratch_types=[
            pltpu.VMEM((G,), jnp.int32),
            pltpu.VMEM((G, OUT_DIM), jnp.float32),
            pltpu.SemaphoreType.DMA,
        ],
    )(idx, table)


def kernel(bbox, x_emb, y_emb, w_emb, h_emb):
    table = jnp.concatenate([x_emb, y_emb, w_emb, h_emb], axis=0)
    idx = bbox.astype(jnp.int32).reshape(TOTAL)
    out = _gather(idx, table)
    return out.reshape(B, S, 4 * OUT_DIM)


# speedup vs baseline: 4.5455x; 1.3044x over previous
import functools

import jax
import jax.numpy as jnp
from jax import lax
from jax.experimental import pallas as pl
from jax.experimental.pallas import tpu as pltpu
from jax.experimental.pallas import tpu_sc as plsc

NUM_BINS = 1000
OUT_DIM = 64
B, S = 4096, 200
TOTAL = B * S * 4
NC, NS = 2, 16
NW = NC * NS
PER_W = TOTAL // NW
G = 128
NBUF = 8
BLK = NBUF * G
NBLK = PER_W // BLK


def _body(idx_hbm, table_hbm, out_hbm, idx_v, rows_v, gsem, ssem, isem):
    wid = lax.axis_index("s") * NC + lax.axis_index("c")
    base = wid * PER_W
    offs = lax.rem(lax.iota(jnp.int32, 16), 4) * NUM_BINS

    def add_offsets(q):
        for m in range(BLK // 16):
            sl = pl.ds(q * BLK + m * 16, 16)
            idx_v[sl] = idx_v[sl] + offs

    pltpu.sync_copy(idx_hbm.at[pl.ds(base, BLK)], idx_v.at[pl.ds(0, BLK)])
    pltpu.sync_copy(idx_hbm.at[pl.ds(base + BLK, BLK)], idx_v.at[pl.ds(BLK, BLK)])

    def block(kb2, q):
        kb = 2 * kb2 + q

        @pl.when(kb2 > 0)
        def _idx_ready():
            pltpu.make_async_copy(idx_hbm.at[pl.ds(0, BLK)],
                                  idx_v.at[pl.ds(q * BLK, BLK)],
                                  isem.at[q]).wait()

        add_offsets(q)

        for j in range(NBUF):
            g0 = kb * NBUF + j

            def _store_gm4():
                s4 = (j + 4) % NBUF
                pltpu.make_async_copy(table_hbm.at[idx_v.at[pl.ds(0, G)]],
                                      rows_v.at[s4], gsem.at[s4]).wait()
                row0 = base + (g0 - 4) * G
                pltpu.async_copy(rows_v.at[s4], out_hbm.at[pl.ds(row0, G)],
                                 ssem.at[s4])
            if j >= 4:
                _store_gm4()
            elif q > 0:
                _store_gm4()
            else:
                pl.when(kb2 > 0)(_store_gm4)

            def _drain():
                pltpu.make_async_copy(rows_v.at[j], out_hbm.at[pl.ds(0, G)],
                                      ssem.at[j]).wait()
            if q > 0:
                _drain()
            else:
                pl.when(kb2 > 0)(_drain)

            src = table_hbm.at[idx_v.at[pl.ds(q * BLK + j * G, G)]]
            pltpu.async_copy(src, rows_v.at[j], gsem.at[j])

            if j == 3:
                def _prefetch():
                    pltpu.async_copy(
                        idx_hbm.at[pl.ds(base + (kb + 1) * BLK, BLK)],
                        idx_v.at[pl.ds((1 - q) * BLK, BLK)], isem.at[1 - q])
                if q == 1:
                    pl.when(kb2 < NBLK // 2 - 1)(_prefetch)
                else:
                    pl.when(kb2 > 0)(_prefetch)

    def step(kb2, carry):
        block(kb2, 0)
        block(kb2, 1)
        return carry

    lax.fori_loop(0, NBLK // 2, step, 0)

    for j in range(4):
        s4 = (j + 4) % NBUF
        pltpu.make_async_copy(table_hbm.at[idx_v.at[pl.ds(0, G)]],
                              rows_v.at[s4], gsem.at[s4]).wait()
        row0 = base + (NBLK * NBUF - 4 + j) * G
        pltpu.async_copy(rows_v.at[s4], out_hbm.at[pl.ds(row0, G)], ssem.at[s4])
    for j in range(NBUF):
        pltpu.make_async_copy(rows_v.at[j], out_hbm.at[pl.ds(0, G)],
                              ssem.at[j]).wait()


@functools.partial(jax.jit, donate_argnums=())
def _gather(idx, table):
    mesh = plsc.VectorSubcoreMesh(core_axis_name="c", subcore_axis_name="s")
    return pl.kernel(
        _body,
        out_type=jax.ShapeDtypeStruct((TOTAL, OUT_DIM), jnp.float32),
        mesh=mesh,
        compiler_params=pltpu.CompilerParams(use_tc_tiling_on_sc=False),
        scratch_types=[
            pltpu.VMEM((2 * BLK,), jnp.int32),
            pltpu.VMEM((NBUF, G, OUT_DIM), jnp.float32),
            pltpu.SemaphoreType.DMA((NBUF,)),
            pltpu.SemaphoreType.DMA((NBUF,)),
            pltpu.SemaphoreType.DMA((2,)),
        ],
    )(idx, table)


def kernel(bbox, x_emb, y_emb, w_emb, h_emb):
    table = jnp.concatenate([x_emb, y_emb, w_emb, h_emb], axis=0)
    idx = bbox.astype(jnp.int32).reshape(TOTAL)
    out = _gather(idx, table)
    return out.reshape(B, S, 4 * OUT_DIM)
